# baseline (speedup 1.0000x reference)
import functools
import math

import jax
import jax.numpy as jnp
from jax.experimental import pallas as pl
from jax.experimental.pallas import tpu as pltpu

_LANE = 128
_HALF_LOG_2PI = 0.5 * math.log(2.0 * 3.141592653)
_BT = 4096
_CHUNK = 512
_RHS_CONTRACT = (((1,), (1,)), ((), ()))


def _policy_kernel(state_ref, opt_ref, act_ref, w1s_ref, w1o_ref, b1_ref,
                   w2_ref, b2_ref, w3_ref, b3_ref, wo_ref, bo_ref, out_ref,
                   xbuf, lterm, sems, *, n_tiles, bt, n_valid):
    def tile_copy(t, slot):
        src = state_ref.at[pl.ds(pl.multiple_of(t * bt, _LANE), bt), :]
        return pltpu.make_async_copy(src, xbuf.at[slot], sems.at[slot])

    tile_copy(0, 0).start()

    def step(i, sse):
        slot = jax.lax.rem(i, 2)

        @pl.when(i + 1 < n_tiles)
        def _():
            tile_copy(i + 1, 1 - slot).start()

        tile_copy(i, slot).wait()

        ev2 = jnp.zeros((1, _CHUNK), jnp.float32)
        for c in range(bt // _CHUNK):
            coff = c * _CHUNK
            off = pl.multiple_of(i * bt + coff, _LANE)
            xs = xbuf[slot, pl.ds(coff, _CHUNK), :].astype(jnp.bfloat16)
            z1 = jax.lax.dot_general(w1s_ref[...], xs, _RHS_CONTRACT,
                                     preferred_element_type=jnp.float32)
            z1 = z1 + w1o_ref[...] * opt_ref[:, pl.ds(off, _CHUNK)] \
                + b1_ref[...]
            h1 = jnp.maximum(z1.astype(jnp.bfloat16), 0)
            z2 = jnp.dot(w2_ref[...], h1, preferred_element_type=jnp.float32)
            h2 = jnp.maximum((z2 + b2_ref[...]).astype(jnp.bfloat16), 0)
            z3 = jnp.dot(w3_ref[...], h2, preferred_element_type=jnp.float32)
            h3 = jnp.tanh(z3 + b3_ref[...]).astype(jnp.bfloat16)
            p = jnp.dot(wo_ref[...], h3,
                        preferred_element_type=jnp.float32) \
                + bo_ref[...]

            mean = jnp.clip(p[0:1, :], -2.0, 2.0)
            p1 = p[1:2, :]
            var = jnp.minimum(jnp.float32(1.0), p1 * p1) + jnp.float32(0.01)
            out_ref[:, pl.ds(off, _CHUNK)] = \
                pl.reciprocal(2.0 * var, approx=False)
            lterm[:, pl.ds(off, _CHUNK)] = \
                -0.5 * jnp.log(var) - _HALF_LOG_2PI

            ev = act_ref[:, pl.ds(off, _CHUNK)] - mean
            if n_valid != n_tiles * bt:
                col = off + jax.lax.broadcasted_iota(
                    jnp.int32, (1, _CHUNK), 1)
                ev = jnp.where(col < n_valid, ev, 0.0)
            ev2 = ev2 + ev * ev
        return sse + jnp.sum(ev2)

    sse = jax.lax.fori_loop(0, n_tiles, step, jnp.float32(0.0))

    out_ref[...] = lterm[...] - sse * out_ref[...]


def kernel(state, action, optim, w1, b1, w2, b2, w3, b3, wo, bo):
    state = jnp.asarray(state, jnp.float32)
    optim = jnp.asarray(optim, jnp.float32).reshape(-1)
    action = jnp.asarray(action, jnp.float32).reshape(-1)

    B, S = state.shape
    H = w1.shape[1]
    A1 = wo.shape[1]

    bt = _BT if B > _BT else max(_LANE, ((B + _LANE - 1) // _LANE) * _LANE)
    Bp = ((B + bt - 1) // bt) * bt
    nt = Bp // bt

    if Bp != B:
        state = jnp.pad(state, ((0, Bp - B), (0, 0)))
        optim = jnp.pad(optim, (0, Bp - B))
        action = jnp.pad(action, (0, Bp - B))
    act_row = action.reshape(1, Bp)
    opt_row = optim.reshape(1, Bp)

    w1s = w1[:S, :].T.astype(jnp.bfloat16)
    w1o = w1[S:, :].T.astype(jnp.float32)
    w2_b = w2.T.astype(jnp.bfloat16)
    w3_b = w3.T.astype(jnp.bfloat16)
    wo_b = wo.T.astype(jnp.bfloat16)
    b1_c = jnp.reshape(b1, (H, 1)).astype(jnp.float32)
    b2_c = jnp.reshape(b2, (H, 1)).astype(jnp.float32)
    b3_c = jnp.reshape(b3, (H, 1)).astype(jnp.float32)
    bo_c = jnp.reshape(bo, (A1, 1)).astype(jnp.float32)

    body = functools.partial(_policy_kernel, n_tiles=nt, bt=bt, n_valid=B)
    vmem = pl.BlockSpec(memory_space=pltpu.MemorySpace.VMEM)
    out = pl.pallas_call(
        body,
        in_specs=[pl.BlockSpec(memory_space=pl.ANY)]
        + [vmem] * 11,
        out_specs=vmem,
        out_shape=jax.ShapeDtypeStruct((1, Bp), jnp.float32),
        scratch_shapes=[
            pltpu.VMEM((2, bt, S), jnp.float32),
            pltpu.VMEM((1, Bp), jnp.float32),
            pltpu.SemaphoreType.DMA((2,)),
        ],
    )(state, opt_row, act_row, w1s, w1o, b1_c,
      w2_b, b2_c, w3_b, b3_c, wo_b, bo_c)

    return out[0, :B]


# speedup vs baseline: 1.1898x; 1.1898x over previous
import functools
import math

import jax
import jax.numpy as jnp
from jax.experimental import pallas as pl
from jax.experimental.pallas import tpu as pltpu

_LANE = 128
_HALF_LOG_2PI = 0.5 * math.log(2.0 * 3.141592653)
_BT = 8192
_RHS_CONTRACT = (((1,), (1,)), ((), ()))


def _policy_kernel(state_ref, opt_ref, act_ref, w1_ref, b1_ref,
                   w2_ref, b2_ref, w3_ref, b3_ref, wo_ref, bo_ref, out_ref,
                   xbuf, obuf, lterm, sems, osems, *,
                   n_tiles, bt, n_valid, n_state):
    def tile_copy(t, slot):
        row0 = pl.ds(pl.multiple_of(t * bt, _LANE), bt)
        return (
            pltpu.make_async_copy(
                state_ref.at[row0, :], xbuf.at[slot], sems.at[slot]),
            pltpu.make_async_copy(
                opt_ref.at[row0, :], obuf.at[slot], osems.at[slot]),
        )

    for cp in tile_copy(0, 0):
        cp.start()

    def step(i, sse):
        slot = jax.lax.rem(i, 2)

        @pl.when(i + 1 < n_tiles)
        def _():
            for cp in tile_copy(i + 1, 1 - slot):
                cp.start()

        for cp in tile_copy(i, slot):
            cp.wait()

        xs = jnp.concatenate([xbuf[slot], obuf[slot]],
                             axis=1).astype(jnp.bfloat16)
        z1 = jax.lax.dot_general(w1_ref[...], xs, _RHS_CONTRACT,
                                 preferred_element_type=jnp.float32)
        off = pl.multiple_of(i * bt, _LANE)
        z1 = z1 + b1_ref[...]
        h1 = jnp.maximum(z1.astype(jnp.bfloat16), 0)
        z2 = jnp.dot(w2_ref[...], h1, preferred_element_type=jnp.float32)
        h2 = jnp.maximum((z2 + b2_ref[...]).astype(jnp.bfloat16), 0)
        z3 = jnp.dot(w3_ref[...], h2, preferred_element_type=jnp.float32)
        h3 = jnp.tanh(z3 + b3_ref[...]).astype(jnp.bfloat16)
        p = jnp.dot(wo_ref[...], h3, preferred_element_type=jnp.float32) \
            + bo_ref[...]

        mean = jnp.clip(p[0:1, :], -2.0, 2.0)
        p1 = p[1:2, :]
        var = jnp.minimum(jnp.float32(1.0), p1 * p1) + jnp.float32(0.01)
        out_ref[:, pl.ds(off, bt)] = pl.reciprocal(2.0 * var, approx=False)
        lterm[:, pl.ds(off, bt)] = -0.5 * jnp.log(var) - _HALF_LOG_2PI

        ev = act_ref[:, pl.ds(off, bt)] - mean
        if n_valid != n_tiles * bt:
            col = off + jax.lax.broadcasted_iota(jnp.int32, (1, bt), 1)
            ev = jnp.where(col < n_valid, ev, 0.0)
        return sse + jnp.sum(ev * ev)

    sse = jax.lax.fori_loop(0, n_tiles, step, jnp.float32(0.0))

    out_ref[...] = lterm[...] - sse * out_ref[...]


def kernel(state, action, optim, w1, b1, w2, b2, w3, b3, wo, bo):
    state = jnp.asarray(state, jnp.float32)
    optim = jnp.asarray(optim, jnp.float32).reshape(-1)
    action = jnp.asarray(action, jnp.float32).reshape(-1)

    B, S = state.shape
    H = w1.shape[1]
    A1 = wo.shape[1]

    bt = _BT if B > _BT else max(_LANE, ((B + _LANE - 1) // _LANE) * _LANE)
    Bp = ((B + bt - 1) // bt) * bt
    nt = Bp // bt

    if Bp != B:
        state = jnp.pad(state, ((0, Bp - B), (0, 0)))
        optim = jnp.pad(optim, (0, Bp - B))
        action = jnp.pad(action, (0, Bp - B))
    act_row = action.reshape(1, Bp)
    opt_col = optim.reshape(Bp, 1)

    w1_b = w1.T.astype(jnp.bfloat16)
    w2_b = w2.T.astype(jnp.bfloat16)
    w3_b = w3.T.astype(jnp.bfloat16)
    wo_b = wo.T.astype(jnp.bfloat16)
    b1_c = jnp.reshape(b1, (H, 1)).astype(jnp.float32)
    b2_c = jnp.reshape(b2, (H, 1)).astype(jnp.float32)
    b3_c = jnp.reshape(b3, (H, 1)).astype(jnp.float32)
    bo_c = jnp.reshape(bo, (A1, 1)).astype(jnp.float32)

    body = functools.partial(_policy_kernel, n_tiles=nt, bt=bt,
                             n_valid=B, n_state=S)
    vmem = pl.BlockSpec(memory_space=pltpu.MemorySpace.VMEM)
    out = pl.pallas_call(
        body,
        in_specs=[pl.BlockSpec(memory_space=pl.ANY)] * 2
        + [vmem] * 9,
        out_specs=vmem,
        out_shape=jax.ShapeDtypeStruct((1, Bp), jnp.float32),
        scratch_shapes=[
            pltpu.VMEM((2, bt, S), jnp.float32),
            pltpu.VMEM((2, bt, 1), jnp.float32),
            pltpu.VMEM((1, Bp), jnp.float32),
            pltpu.SemaphoreType.DMA((2,)),
            pltpu.SemaphoreType.DMA((2,)),
        ],
    )(state, opt_col, act_row, w1_b, b1_c,
      w2_b, b2_c, w3_b, b3_c, wo_b, bo_c)

    return out[0, :B]


# speedup vs baseline: 1.3523x; 1.1366x over previous
import functools
import math

import jax
import jax.numpy as jnp
from jax.experimental import pallas as pl
from jax.experimental.pallas import tpu as pltpu

_LANE = 128
_HALF_LOG_2PI = 0.5 * math.log(2.0 * 3.141592653)
_BT = 16384
_RHS_CONTRACT = (((1,), (1,)), ((), ()))


def _policy_kernel(state_ref, opt_ref, act_ref, w1s_ref, w1o_ref, b1_ref,
                   w2_ref, b2_ref, w3_ref, b3_ref, wo_ref, bo_ref, out_ref,
                   xbuf, lterm, sems, *, n_tiles, bt, n_valid):
    def tile_copy(t, slot):
        src = state_ref.at[pl.ds(pl.multiple_of(t * bt, _LANE), bt), :]
        return pltpu.make_async_copy(src, xbuf.at[slot], sems.at[slot])

    tile_copy(0, 0).start()

    def step(i, sse):
        slot = jax.lax.rem(i, 2)

        @pl.when(i + 1 < n_tiles)
        def _():
            tile_copy(i + 1, 1 - slot).start()

        tile_copy(i, slot).wait()

        xs = xbuf[slot].astype(jnp.bfloat16)
        z1 = jax.lax.dot_general(w1s_ref[...], xs, _RHS_CONTRACT,
                                 preferred_element_type=jnp.float32)
        off = pl.multiple_of(i * bt, _LANE)
        z1 = z1 + w1o_ref[...] * opt_ref[:, pl.ds(off, bt)] + b1_ref[...]
        h1 = jnp.maximum(z1.astype(jnp.bfloat16), 0)
        z2 = jnp.dot(w2_ref[...], h1, preferred_element_type=jnp.float32)
        h2 = jnp.maximum((z2 + b2_ref[...]).astype(jnp.bfloat16), 0)
        z3 = jnp.dot(w3_ref[...], h2, preferred_element_type=jnp.float32)
        h3 = jnp.tanh(z3 + b3_ref[...]).astype(jnp.bfloat16)
        p = jnp.dot(wo_ref[...], h3, preferred_element_type=jnp.float32) \
            + bo_ref[...]

        mean = jnp.clip(p[0:1, :], -2.0, 2.0)
        p1 = p[1:2, :]
        var = jnp.minimum(jnp.float32(1.0), p1 * p1) + jnp.float32(0.01)
        out_ref[:, pl.ds(off, bt)] = pl.reciprocal(2.0 * var, approx=False)
        lterm[:, pl.ds(off, bt)] = -0.5 * jnp.log(var) - _HALF_LOG_2PI

        ev = act_ref[:, pl.ds(off, bt)] - mean
        if n_valid != n_tiles * bt:
            col = off + jax.lax.broadcasted_iota(jnp.int32, (1, bt), 1)
            ev = jnp.where(col < n_valid, ev, 0.0)
        return sse + jnp.sum(ev * ev)

    sse = jax.lax.fori_loop(0, n_tiles, step, jnp.float32(0.0))

    out_ref[...] = lterm[...] - sse * out_ref[...]


def kernel(state, action, optim, w1, b1, w2, b2, w3, b3, wo, bo):
    state = jnp.asarray(state, jnp.float32)
    optim = jnp.asarray(optim, jnp.float32).reshape(-1)
    action = jnp.asarray(action, jnp.float32).reshape(-1)

    B, S = state.shape
    H = w1.shape[1]
    A1 = wo.shape[1]

    bt = _BT if B > _BT else max(_LANE, ((B + _LANE - 1) // _LANE) * _LANE)
    Bp = ((B + bt - 1) // bt) * bt
    nt = Bp // bt

    if Bp != B:
        state = jnp.pad(state, ((0, Bp - B), (0, 0)))
        optim = jnp.pad(optim, (0, Bp - B))
        action = jnp.pad(action, (0, Bp - B))
    act_row = action.reshape(1, Bp)
    opt_row = optim.reshape(1, Bp)

    w1s = w1[:S, :].T.astype(jnp.bfloat16)
    w1o = w1[S:, :].T.astype(jnp.float32)
    w2_b = w2.T.astype(jnp.bfloat16)
    w3_b = w3.T.astype(jnp.bfloat16)
    wo_b = wo.T.astype(jnp.bfloat16)
    b1_c = jnp.reshape(b1, (H, 1)).astype(jnp.float32)
    b2_c = jnp.reshape(b2, (H, 1)).astype(jnp.float32)
    b3_c = jnp.reshape(b3, (H, 1)).astype(jnp.float32)
    bo_c = jnp.reshape(bo, (A1, 1)).astype(jnp.float32)

    body = functools.partial(_policy_kernel, n_tiles=nt, bt=bt, n_valid=B)
    vmem = pl.BlockSpec(memory_space=pltpu.MemorySpace.VMEM)
    out = pl.pallas_call(
        body,
        in_specs=[pl.BlockSpec(memory_space=pl.ANY)]
        + [vmem] * 11,
        out_specs=vmem,
        out_shape=jax.ShapeDtypeStruct((1, Bp), jnp.float32),
        scratch_shapes=[
            pltpu.VMEM((2, bt, S), jnp.float32),
            pltpu.VMEM((1, Bp), jnp.float32),
            pltpu.SemaphoreType.DMA((2,)),
        ],
    )(state, opt_row, act_row, w1s, w1o, b1_c,
      w2_b, b2_c, w3_b, b3_c, wo_b, bo_c)

    return out[0, :B]


# speedup vs baseline: 1.3615x; 1.0068x over previous
import functools
import math

import jax
import jax.numpy as jnp
from jax.experimental import pallas as pl
from jax.experimental.pallas import tpu as pltpu

_LANE = 128
_HALF_LOG_2PI = 0.5 * math.log(2.0 * 3.141592653)
_BT = 16384
_RHS_CONTRACT = (((1,), (1,)), ((), ()))


def _policy_kernel(xs_ref, opt_ref, act_ref, w1s_ref, w1o_ref, b1_ref,
                   w2_ref, b2_ref, w3_ref, b3_ref, wo_ref, bo_ref, out_ref,
                   var_buf, sse_acc, *, n_tiles, bt, n_valid):
    i = pl.program_id(0)

    @pl.when(i == 0)
    def _():
        sse_acc[...] = jnp.zeros_like(sse_acc)

    xs = xs_ref[...].astype(jnp.bfloat16)
    z1 = jax.lax.dot_general(w1s_ref[...], xs, _RHS_CONTRACT,
                             preferred_element_type=jnp.float32)
    z1 = z1 + w1o_ref[...] * opt_ref[...] + b1_ref[...]
    h1 = jnp.maximum(z1.astype(jnp.bfloat16), 0)
    z2 = jnp.dot(w2_ref[...], h1, preferred_element_type=jnp.float32)
    h2 = jnp.maximum((z2 + b2_ref[...]).astype(jnp.bfloat16), 0)
    z3 = jnp.dot(w3_ref[...], h2, preferred_element_type=jnp.float32)
    h3 = jnp.tanh(z3 + b3_ref[...]).astype(jnp.bfloat16)
    p = jnp.dot(wo_ref[...], h3, preferred_element_type=jnp.float32) \
        + bo_ref[...]

    mean = jnp.clip(p[0:1, :], -2.0, 2.0)
    p1 = p[1:2, :]
    var = jnp.minimum(jnp.float32(1.0), p1 * p1) + jnp.float32(0.01)

    col = i * bt + jax.lax.broadcasted_iota(jnp.int32, (1, bt), 1)
    ev = jnp.where(col < n_valid, act_ref[...] - mean, 0.0)
    sse_acc[...] += jnp.sum(ev * ev, keepdims=True)

    off = pl.multiple_of(i * bt, _LANE)
    var_buf[:, pl.ds(off, bt)] = var

    @pl.when(i == n_tiles - 1)
    def _():
        v = var_buf[...]
        out_ref[...] = (-sse_acc[...]) / (2.0 * v) \
            - 0.5 * jnp.log(v) - _HALF_LOG_2PI


def kernel(state, action, optim, w1, b1, w2, b2, w3, b3, wo, bo):
    state = jnp.asarray(state, jnp.float32)
    optim = jnp.asarray(optim, jnp.float32).reshape(-1)
    action = jnp.asarray(action, jnp.float32).reshape(-1)

    B, S = state.shape
    H = w1.shape[1]
    A1 = wo.shape[1]

    bt = _BT if B > _BT else max(_LANE, ((B + _LANE - 1) // _LANE) * _LANE)
    Bp = ((B + bt - 1) // bt) * bt
    nt = Bp // bt

    if Bp != B:
        state = jnp.pad(state, ((0, Bp - B), (0, 0)))
        optim = jnp.pad(optim, (0, Bp - B))
        action = jnp.pad(action, (0, Bp - B))
    act_row = action.reshape(1, Bp)
    opt_row = optim.reshape(1, Bp)

    w1s = w1[:S, :].T.astype(jnp.bfloat16)
    w1o = w1[S:, :].T.astype(jnp.float32)
    w2_b = w2.T.astype(jnp.bfloat16)
    w3_b = w3.T.astype(jnp.bfloat16)
    wo_b = wo.T.astype(jnp.bfloat16)
    b1_c = jnp.reshape(b1, (H, 1)).astype(jnp.float32)
    b2_c = jnp.reshape(b2, (H, 1)).astype(jnp.float32)
    b3_c = jnp.reshape(b3, (H, 1)).astype(jnp.float32)
    bo_c = jnp.reshape(bo, (A1, 1)).astype(jnp.float32)

    body = functools.partial(_policy_kernel, n_tiles=nt, bt=bt, n_valid=B)
    out = pl.pallas_call(
        body,
        grid=(nt,),
        in_specs=[
            pl.BlockSpec((bt, S), lambda i: (i, 0)),
            pl.BlockSpec((1, bt), lambda i: (0, i)),
            pl.BlockSpec((1, bt), lambda i: (0, i)),
            pl.BlockSpec((H, S), lambda i: (0, 0)),
            pl.BlockSpec((H, 1), lambda i: (0, 0)),
            pl.BlockSpec((H, 1), lambda i: (0, 0)),
            pl.BlockSpec((H, H), lambda i: (0, 0)),
            pl.BlockSpec((H, 1), lambda i: (0, 0)),
            pl.BlockSpec((H, H), lambda i: (0, 0)),
            pl.BlockSpec((H, 1), lambda i: (0, 0)),
            pl.BlockSpec((A1, H), lambda i: (0, 0)),
            pl.BlockSpec((A1, 1), lambda i: (0, 0)),
        ],
        out_specs=pl.BlockSpec((1, Bp), lambda i: (0, 0)),
        out_shape=jax.ShapeDtypeStruct((1, Bp), jnp.float32),
        scratch_shapes=[
            pltpu.VMEM((1, Bp), jnp.float32),
            pltpu.VMEM((1, 1), jnp.float32),
        ],
        compiler_params=pltpu.CompilerParams(
            dimension_semantics=("arbitrary",)),
    )(state, opt_row, act_row, w1s, w1o, b1_c,
      w2_b, b2_c, w3_b, b3_c, wo_b, bo_c)

    return out[0, :B]
